# initial kernel scaffold (unmeasured)
import jax
import jax.numpy as jnp
from jax import lax
from jax.experimental import pallas as pl
from jax.experimental.pallas import tpu as pltpu

N_DEV = 32
ROWS = 4096
COLS = 8192
CHUNK = ROWS // N_DEV

_PLANE = [(0, 0), (1, 0), (1, 1), (0, 1), (0, 2), (1, 2), (1, 3), (0, 3)]
_SNAKE = [(x, y, z) for z in range(4) for (x, y) in _PLANE]
_C16 = [(0, 0), (0, 1), (0, 2), (0, 3), (1, 3), (1, 2), (1, 1), (2, 1),
        (2, 2), (2, 3), (3, 3), (3, 2), (3, 1), (3, 0), (2, 0), (1, 0)]
_CYCLE = [(0, y, z) for (y, z) in _C16] + \
         [(1, y, z) for (y, z) in reversed(_C16)]
assert len(set(_CYCLE)) == N_DEV
for _i in range(N_DEV):
    _a, _b = _CYCLE[_i], _CYCLE[(_i + 1) % N_DEV]
    assert sum(abs(p - q) for p, q in zip(_a, _b)) == 1, (_a, _b)
_PERM = [_SNAKE.index(c) for c in _CYCLE]
_INV = [0] * N_DEV
for _p, _l in enumerate(_PERM):
    _INV[_l] = _p
_NEXT = [_PERM[(_INV[l] + 1) % N_DEV] for l in range(N_DEV)]
_PREV = [_PERM[(_INV[l] - 1) % N_DEV] for l in range(N_DEV)]

_LOGICAL = pl.DeviceIdType.LOGICAL


def _body(meta_ref, sc_ref, partial_ref, out_ref,
          acc_buf, loc_buf, gat_buf,
          rs_send_sems, rs_recv_sems, ag_send_sems, ag_recv_sems,
          copy_sems, store_sem, rs_credit, ag_credit):
    my_pos = meta_ref[0]
    right = meta_ref[1]
    left = meta_ref[2]
    sc = sc_ref[0]

    barrier = pltpu.get_barrier_semaphore()
    pl.semaphore_signal(barrier, inc=1, device_id=left,
                        device_id_type=_LOGICAL)
    pl.semaphore_signal(barrier, inc=1, device_id=right,
                        device_id_type=_LOGICAL)
    pl.semaphore_wait(barrier, 2)

    def rows(i):
        return pl.ds(i * CHUNK, CHUNK)

    for t in range(N_DEV - 1):
        a, b = (t - 1) % 2, t % 2
        send_chunk = (my_pos - t) % N_DEV
        recv_chunk = (my_pos - t - 1) % N_DEV
        if t >= 2:
            pl.semaphore_wait(rs_credit, 1)
        src = partial_ref.at[rows(send_chunk), :] if t == 0 else acc_buf.at[a]
        rdma = pltpu.make_async_remote_copy(
            src_ref=src,
            dst_ref=acc_buf.at[b],
            send_sem=rs_send_sems.at[b],
            recv_sem=rs_recv_sems.at[b],
            device_id=right,
            device_id_type=_LOGICAL,
        )
        rdma.start()
        cp = pltpu.make_async_copy(
            partial_ref.at[rows(recv_chunk), :], loc_buf.at[b],
            copy_sems.at[b])
        cp.start()
        rdma.wait()
        if 1 <= t <= N_DEV - 3:
            pl.semaphore_signal(rs_credit, inc=1, device_id=left,
                                device_id_type=_LOGICAL)
        cp.wait()
        acc_buf[b] = acc_buf[b] + loc_buf[b]

    own = (my_pos + 1) % N_DEV
    y = acc_buf[(N_DEV - 2) % 2] * sc
    gat_buf[0] = y * (1.0 / (1.0 + jnp.exp(-y)))
    st = pltpu.make_async_copy(gat_buf.at[0], out_ref.at[rows(own), :],
                               store_sem)
    st.start()
    st.wait()

    for t in range(N_DEV - 1):
        a, b = t % 2, (t + 1) % 2
        recv_chunk = (my_pos - t) % N_DEV
        if t >= 2:
            pl.semaphore_wait(ag_credit, 1)
        rdma = pltpu.make_async_remote_copy(
            src_ref=gat_buf.at[a],
            dst_ref=gat_buf.at[b],
            send_sem=ag_send_sems.at[b],
            recv_sem=ag_recv_sems.at[b],
            device_id=right,
            device_id_type=_LOGICAL,
        )
        rdma.start()
        rdma.wait()
        if 1 <= t <= N_DEV - 3:
            pl.semaphore_signal(ag_credit, inc=1, device_id=left,
                                device_id_type=_LOGICAL)
        st = pltpu.make_async_copy(
            gat_buf.at[b], out_ref.at[rows(recv_chunk), :], store_sem)
        st.start()
        st.wait()


def kernel(x, w_mat, scale_x, scale_w):
    partial = jnp.dot(x, w_mat, preferred_element_type=jnp.float32)
    sc = (scale_x * scale_w).astype(jnp.float32)
    d = lax.axis_index("i")
    meta = jnp.stack([
        jnp.asarray(_INV, dtype=jnp.int32)[d],
        jnp.asarray(_NEXT, dtype=jnp.int32)[d],
        jnp.asarray(_PREV, dtype=jnp.int32)[d],
    ]).astype(jnp.int32)

    return pl.pallas_call(
        _body,
        out_shape=jax.ShapeDtypeStruct((ROWS, COLS), jnp.float32),
        in_specs=[
            pl.BlockSpec(memory_space=pltpu.SMEM),
            pl.BlockSpec(memory_space=pltpu.SMEM),
            pl.BlockSpec(memory_space=pltpu.ANY),
        ],
        out_specs=pl.BlockSpec(memory_space=pltpu.ANY),
        scratch_shapes=[
            pltpu.VMEM((2, CHUNK, COLS), jnp.float32),
            pltpu.VMEM((2, CHUNK, COLS), jnp.float32),
            pltpu.VMEM((2, CHUNK, COLS), jnp.float32),
            pltpu.SemaphoreType.DMA((2,)),
            pltpu.SemaphoreType.DMA((2,)),
            pltpu.SemaphoreType.DMA((2,)),
            pltpu.SemaphoreType.DMA((2,)),
            pltpu.SemaphoreType.DMA((2,)),
            pltpu.SemaphoreType.DMA,
            pltpu.SemaphoreType.REGULAR,
            pltpu.SemaphoreType.REGULAR,
        ],
        compiler_params=pltpu.CompilerParams(collective_id=0),
    )(meta, sc, partial)


# baseline (device time: 3120561 ns/iter reference)
import jax
import jax.numpy as jnp
from jax import lax
from jax.experimental import pallas as pl
from jax.experimental.pallas import tpu as pltpu

N_DEV = 32
ROWS = 4096
COLS = 8192
CHUNK = ROWS // N_DEV

_PLANE = [(0, 0), (1, 0), (1, 1), (0, 1), (0, 2), (1, 2), (1, 3), (0, 3)]
_SNAKE = [(x, y, z) for z in range(4) for (x, y) in _PLANE]
_C16 = [(0, 0), (0, 1), (0, 2), (0, 3), (1, 3), (1, 2), (1, 1), (2, 1),
        (2, 2), (2, 3), (3, 3), (3, 2), (3, 1), (3, 0), (2, 0), (1, 0)]
_CYCLE = [(0, y, z) for (y, z) in _C16] + \
         [(1, y, z) for (y, z) in reversed(_C16)]
assert len(set(_CYCLE)) == N_DEV
for _i in range(N_DEV):
    _a, _b = _CYCLE[_i], _CYCLE[(_i + 1) % N_DEV]
    assert sum(abs(p - q) for p, q in zip(_a, _b)) == 1, (_a, _b)
_PERM = [_SNAKE.index(c) for c in _CYCLE]
_INV = [0] * N_DEV
for _p, _l in enumerate(_PERM):
    _INV[_l] = _p
_NEXT = [_PERM[(_INV[l] + 1) % N_DEV] for l in range(N_DEV)]
_PREV = [_PERM[(_INV[l] - 1) % N_DEV] for l in range(N_DEV)]

_LOGICAL = pl.DeviceIdType.LOGICAL


def _body(meta_ref, sc_ref, partial_ref, out_ref,
          acc_buf, loc_buf, gat_buf,
          rs_send_sems, rs_recv_sems, ag_send_sems, ag_recv_sems,
          copy_sems, store_sem, rs_credit, ag_credit):
    my_pos = meta_ref[0]
    right = meta_ref[1]
    left = meta_ref[2]
    sc = sc_ref[0]

    barrier = pltpu.get_barrier_semaphore()
    pl.semaphore_signal(barrier, inc=1, device_id=left,
                        device_id_type=_LOGICAL)
    pl.semaphore_signal(barrier, inc=1, device_id=right,
                        device_id_type=_LOGICAL)
    pl.semaphore_wait(barrier, 2)

    def rows(i):
        return pl.ds(i * CHUNK, CHUNK)

    for t in range(N_DEV - 1):
        a, b = (t - 1) % 2, t % 2
        send_chunk = (my_pos - t) % N_DEV
        recv_chunk = (my_pos - t - 1) % N_DEV
        if t >= 2:
            pl.semaphore_wait(rs_credit, 1)
        src = partial_ref.at[rows(send_chunk), :] if t == 0 else acc_buf.at[a]
        rdma = pltpu.make_async_remote_copy(
            src_ref=src,
            dst_ref=acc_buf.at[b],
            send_sem=rs_send_sems.at[b],
            recv_sem=rs_recv_sems.at[b],
            device_id=right,
            device_id_type=_LOGICAL,
        )
        rdma.start()
        cp = pltpu.make_async_copy(
            partial_ref.at[rows(recv_chunk), :], loc_buf.at[b],
            copy_sems.at[b])
        cp.start()
        rdma.wait()
        if 1 <= t <= N_DEV - 3:
            pl.semaphore_signal(rs_credit, inc=1, device_id=left,
                                device_id_type=_LOGICAL)
        cp.wait()
        acc_buf[b] = acc_buf[b] + loc_buf[b]

    own = (my_pos + 1) % N_DEV
    y = acc_buf[(N_DEV - 2) % 2] * sc
    gat_buf[0] = y * (1.0 / (1.0 + jnp.exp(-y)))
    st = pltpu.make_async_copy(gat_buf.at[0], out_ref.at[rows(own), :],
                               store_sem)
    st.start()
    st.wait()

    for t in range(N_DEV - 1):
        a, b = t % 2, (t + 1) % 2
        recv_chunk = (my_pos - t) % N_DEV
        if t >= 2:
            pl.semaphore_wait(ag_credit, 1)
        rdma = pltpu.make_async_remote_copy(
            src_ref=gat_buf.at[a],
            dst_ref=gat_buf.at[b],
            send_sem=ag_send_sems.at[b],
            recv_sem=ag_recv_sems.at[b],
            device_id=right,
            device_id_type=_LOGICAL,
        )
        rdma.start()
        rdma.wait()
        if 1 <= t <= N_DEV - 3:
            pl.semaphore_signal(ag_credit, inc=1, device_id=left,
                                device_id_type=_LOGICAL)
        st = pltpu.make_async_copy(
            gat_buf.at[b], out_ref.at[rows(recv_chunk), :], store_sem)
        st.start()
        st.wait()


def kernel(x, w_mat, scale_x, scale_w):
    partial = jnp.dot(x, w_mat, preferred_element_type=jnp.float32)
    sc = (scale_x * scale_w).astype(jnp.float32)
    d = lax.axis_index("i")
    meta = jnp.stack([
        jnp.asarray(_INV, dtype=jnp.int32)[d],
        jnp.asarray(_NEXT, dtype=jnp.int32)[d],
        jnp.asarray(_PREV, dtype=jnp.int32)[d],
    ]).astype(jnp.int32)

    return pl.pallas_call(
        _body,
        out_shape=jax.ShapeDtypeStruct((ROWS, COLS), jnp.float32),
        in_specs=[
            pl.BlockSpec(memory_space=pltpu.SMEM),
            pl.BlockSpec(memory_space=pltpu.SMEM),
            pl.BlockSpec(memory_space=pl.ANY),
        ],
        out_specs=pl.BlockSpec(memory_space=pl.ANY),
        scratch_shapes=[
            pltpu.VMEM((2, CHUNK, COLS), jnp.float32),
            pltpu.VMEM((2, CHUNK, COLS), jnp.float32),
            pltpu.VMEM((2, CHUNK, COLS), jnp.float32),
            pltpu.SemaphoreType.DMA((2,)),
            pltpu.SemaphoreType.DMA((2,)),
            pltpu.SemaphoreType.DMA((2,)),
            pltpu.SemaphoreType.DMA((2,)),
            pltpu.SemaphoreType.DMA((2,)),
            pltpu.SemaphoreType.DMA,
            pltpu.SemaphoreType.REGULAR,
            pltpu.SemaphoreType.REGULAR,
        ],
        compiler_params=pltpu.CompilerParams(collective_id=0),
    )(meta, sc, partial)


# device time: 1749627 ns/iter; 1.7836x vs baseline; 1.7836x over previous
import jax
import jax.numpy as jnp
from jax import lax
from jax.experimental import pallas as pl
from jax.experimental.pallas import tpu as pltpu

N_DEV = 32
ROWS = 4096
COLS = 8192
HALF = COLS // 2
CHUNK = ROWS // N_DEV

_PLANE = [(0, 0), (1, 0), (1, 1), (0, 1), (0, 2), (1, 2), (1, 3), (0, 3)]
_SNAKE = [(x, y, z) for z in range(4) for (x, y) in _PLANE]
_C16 = [(0, 0), (0, 1), (0, 2), (0, 3), (1, 3), (1, 2), (1, 1), (2, 1),
        (2, 2), (2, 3), (3, 3), (3, 2), (3, 1), (3, 0), (2, 0), (1, 0)]
_CYCLE = [(0, y, z) for (y, z) in _C16] + \
         [(1, y, z) for (y, z) in reversed(_C16)]
assert len(set(_CYCLE)) == N_DEV
for _i in range(N_DEV):
    _a, _b = _CYCLE[_i], _CYCLE[(_i + 1) % N_DEV]
    assert sum(abs(p - q) for p, q in zip(_a, _b)) == 1, (_a, _b)
_PERM = [_SNAKE.index(c) for c in _CYCLE]
_INV = [0] * N_DEV
for _p, _l in enumerate(_PERM):
    _INV[_l] = _p
_NEXT = [_PERM[(_INV[l] + 1) % N_DEV] for l in range(N_DEV)]
_PREV = [_PERM[(_INV[l] - 1) % N_DEV] for l in range(N_DEV)]

_LOGICAL = pl.DeviceIdType.LOGICAL


def _body(meta_ref, sc_ref, partial_ref, out_ref,
          acc_f, acc_b, loc_f, loc_b, gat_f, gat_b,
          rs_send_f, rs_recv_f, rs_send_b, rs_recv_b,
          ag_send_f, ag_recv_f, ag_send_b, ag_recv_b,
          copy_sems_f, copy_sems_b, store_sem,
          rs_credit_f, rs_credit_b, ag_credit_f, ag_credit_b):
    my_pos = meta_ref[0]
    right = meta_ref[1]
    left = meta_ref[2]
    sc = sc_ref[0]

    barrier = pltpu.get_barrier_semaphore()
    pl.semaphore_signal(barrier, inc=1, device_id=left,
                        device_id_type=_LOGICAL)
    pl.semaphore_signal(barrier, inc=1, device_id=right,
                        device_id_type=_LOGICAL)
    pl.semaphore_wait(barrier, 2)

    def rows(i):
        return pl.ds(i * CHUNK, CHUNK)

    fcols = pl.ds(0, HALF)
    bcols = pl.ds(HALF, HALF)

    for t in range(N_DEV - 1):
        a, b = (t - 1) % 2, t % 2
        send_f = (my_pos - t) % N_DEV
        recv_f = (my_pos - t - 1) % N_DEV
        send_b = (my_pos + t) % N_DEV
        recv_b = (my_pos + t + 1) % N_DEV
        if t >= 2:
            pl.semaphore_wait(rs_credit_f, 1)
        src_f = (partial_ref.at[rows(send_f), fcols] if t == 0
                 else acc_f.at[a])
        rdma_f = pltpu.make_async_remote_copy(
            src_ref=src_f, dst_ref=acc_f.at[b],
            send_sem=rs_send_f.at[b], recv_sem=rs_recv_f.at[b],
            device_id=right, device_id_type=_LOGICAL)
        rdma_f.start()
        if t >= 2:
            pl.semaphore_wait(rs_credit_b, 1)
        src_b = (partial_ref.at[rows(send_b), bcols] if t == 0
                 else acc_b.at[a])
        rdma_b = pltpu.make_async_remote_copy(
            src_ref=src_b, dst_ref=acc_b.at[b],
            send_sem=rs_send_b.at[b], recv_sem=rs_recv_b.at[b],
            device_id=left, device_id_type=_LOGICAL)
        rdma_b.start()
        cp_f = pltpu.make_async_copy(
            partial_ref.at[rows(recv_f), fcols], loc_f.at[b],
            copy_sems_f.at[b])
        cp_f.start()
        cp_b = pltpu.make_async_copy(
            partial_ref.at[rows(recv_b), bcols], loc_b.at[b],
            copy_sems_b.at[b])
        cp_b.start()
        rdma_f.wait()
        if 1 <= t <= N_DEV - 3:
            pl.semaphore_signal(rs_credit_f, inc=1, device_id=left,
                                device_id_type=_LOGICAL)
        rdma_b.wait()
        if 1 <= t <= N_DEV - 3:
            pl.semaphore_signal(rs_credit_b, inc=1, device_id=right,
                                device_id_type=_LOGICAL)
        cp_f.wait()
        acc_f[b] = acc_f[b] + loc_f[b]
        cp_b.wait()
        acc_b[b] = acc_b[b] + loc_b[b]

    last = (N_DEV - 2) % 2
    own_f = (my_pos + 1) % N_DEV
    own_b = (my_pos - 1) % N_DEV
    y = acc_f[last] * sc
    gat_f[0] = y * (1.0 / (1.0 + jnp.exp(-y)))
    y = acc_b[last] * sc
    gat_b[0] = y * (1.0 / (1.0 + jnp.exp(-y)))
    st = pltpu.make_async_copy(gat_f.at[0], out_ref.at[rows(own_f), fcols],
                               store_sem)
    st.start()
    st.wait()
    st = pltpu.make_async_copy(gat_b.at[0], out_ref.at[rows(own_b), bcols],
                               store_sem)
    st.start()
    st.wait()

    for t in range(N_DEV - 1):
        a, b = t % 2, (t + 1) % 2
        recv_f = (my_pos - t) % N_DEV
        recv_b = (my_pos + t) % N_DEV
        if t >= 2:
            pl.semaphore_wait(ag_credit_f, 1)
        rdma_f = pltpu.make_async_remote_copy(
            src_ref=gat_f.at[a], dst_ref=gat_f.at[b],
            send_sem=ag_send_f.at[b], recv_sem=ag_recv_f.at[b],
            device_id=right, device_id_type=_LOGICAL)
        rdma_f.start()
        if t >= 2:
            pl.semaphore_wait(ag_credit_b, 1)
        rdma_b = pltpu.make_async_remote_copy(
            src_ref=gat_b.at[a], dst_ref=gat_b.at[b],
            send_sem=ag_send_b.at[b], recv_sem=ag_recv_b.at[b],
            device_id=left, device_id_type=_LOGICAL)
        rdma_b.start()
        rdma_f.wait()
        if 1 <= t <= N_DEV - 3:
            pl.semaphore_signal(ag_credit_f, inc=1, device_id=left,
                                device_id_type=_LOGICAL)
        rdma_b.wait()
        if 1 <= t <= N_DEV - 3:
            pl.semaphore_signal(ag_credit_b, inc=1, device_id=right,
                                device_id_type=_LOGICAL)
        st = pltpu.make_async_copy(
            gat_f.at[b], out_ref.at[rows(recv_f), fcols], store_sem)
        st.start()
        st.wait()
        st = pltpu.make_async_copy(
            gat_b.at[b], out_ref.at[rows(recv_b), bcols], store_sem)
        st.start()
        st.wait()


def kernel(x, w_mat, scale_x, scale_w):
    partial = jnp.dot(x, w_mat, preferred_element_type=jnp.float32)
    sc = (scale_x * scale_w).astype(jnp.float32)
    d = lax.axis_index("i")
    meta = jnp.stack([
        jnp.asarray(_INV, dtype=jnp.int32)[d],
        jnp.asarray(_NEXT, dtype=jnp.int32)[d],
        jnp.asarray(_PREV, dtype=jnp.int32)[d],
    ]).astype(jnp.int32)

    return pl.pallas_call(
        _body,
        out_shape=jax.ShapeDtypeStruct((ROWS, COLS), jnp.float32),
        in_specs=[
            pl.BlockSpec(memory_space=pltpu.SMEM),
            pl.BlockSpec(memory_space=pltpu.SMEM),
            pl.BlockSpec(memory_space=pl.ANY),
        ],
        out_specs=pl.BlockSpec(memory_space=pl.ANY),
        scratch_shapes=[
            pltpu.VMEM((2, CHUNK, HALF), jnp.float32),
            pltpu.VMEM((2, CHUNK, HALF), jnp.float32),
            pltpu.VMEM((2, CHUNK, HALF), jnp.float32),
            pltpu.VMEM((2, CHUNK, HALF), jnp.float32),
            pltpu.VMEM((2, CHUNK, HALF), jnp.float32),
            pltpu.VMEM((2, CHUNK, HALF), jnp.float32),
            pltpu.SemaphoreType.DMA((2,)),
            pltpu.SemaphoreType.DMA((2,)),
            pltpu.SemaphoreType.DMA((2,)),
            pltpu.SemaphoreType.DMA((2,)),
            pltpu.SemaphoreType.DMA((2,)),
            pltpu.SemaphoreType.DMA((2,)),
            pltpu.SemaphoreType.DMA((2,)),
            pltpu.SemaphoreType.DMA((2,)),
            pltpu.SemaphoreType.DMA((2,)),
            pltpu.SemaphoreType.DMA((2,)),
            pltpu.SemaphoreType.DMA,
            pltpu.SemaphoreType.REGULAR,
            pltpu.SemaphoreType.REGULAR,
            pltpu.SemaphoreType.REGULAR,
            pltpu.SemaphoreType.REGULAR,
        ],
        compiler_params=pltpu.CompilerParams(collective_id=0),
    )(meta, sc, partial)


# device time: 992572 ns/iter; 3.1439x vs baseline; 1.7627x over previous
import jax
import jax.numpy as jnp
from jax import lax
from jax.experimental import pallas as pl
from jax.experimental.pallas import tpu as pltpu

N_DEV = 32
ROWS = 4096
COLS = 8192
HALF = COLS // 2
CHUNK = ROWS // N_DEV

_PLANE = [(0, 0), (1, 0), (1, 1), (0, 1), (0, 2), (1, 2), (1, 3), (0, 3)]
_SNAKE = [(x, y, z) for z in range(4) for (x, y) in _PLANE]
_C16 = [(0, 0), (0, 1), (0, 2), (0, 3), (1, 3), (1, 2), (1, 1), (2, 1),
        (2, 2), (2, 3), (3, 3), (3, 2), (3, 1), (3, 0), (2, 0), (1, 0)]
_CYCLE = [(0, y, z) for (y, z) in _C16] + \
         [(1, y, z) for (y, z) in reversed(_C16)]
assert len(set(_CYCLE)) == N_DEV
for _i in range(N_DEV):
    _a, _b = _CYCLE[_i], _CYCLE[(_i + 1) % N_DEV]
    assert sum(abs(p - q) for p, q in zip(_a, _b)) == 1, (_a, _b)
_PERM = [_SNAKE.index(c) for c in _CYCLE]
_INV = [0] * N_DEV
for _p, _l in enumerate(_PERM):
    _INV[_l] = _p
_NEXT = [_PERM[(_INV[l] + 1) % N_DEV] for l in range(N_DEV)]
_PREV = [_PERM[(_INV[l] - 1) % N_DEV] for l in range(N_DEV)]

_LOGICAL = pl.DeviceIdType.LOGICAL
_F16 = jnp.bfloat16


def _body(meta_ref, sc_ref, partial_ref, out_ref,
          loc_f, loc_b, rcv_f, rcv_b, snd_f, snd_b,
          ag_f, ag_b, stg_f, stg_b,
          rs_send_f, rs_recv_f, rs_send_b, rs_recv_b,
          ag_send_f, ag_recv_f, ag_send_b, ag_recv_b,
          copy_f, copy_b, store_f, store_b,
          rs_credit_f, rs_credit_b, ag_credit_f, ag_credit_b):
    my_pos = meta_ref[0]
    right = meta_ref[1]
    left = meta_ref[2]
    sc = sc_ref[0]

    barrier = pltpu.get_barrier_semaphore()
    pl.semaphore_signal(barrier, inc=1, device_id=left,
                        device_id_type=_LOGICAL)
    pl.semaphore_signal(barrier, inc=1, device_id=right,
                        device_id_type=_LOGICAL)
    pl.semaphore_wait(barrier, 2)

    def rows(i):
        return pl.ds(i * CHUNK, CHUNK)

    fcols = pl.ds(0, HALF)
    bcols = pl.ds(HALF, HALF)

    cp0f = pltpu.make_async_copy(
        partial_ref.at[rows(my_pos), fcols], loc_f.at[0], copy_f.at[0])
    cp0f.start()
    cp0b = pltpu.make_async_copy(
        partial_ref.at[rows(my_pos), bcols], loc_b.at[0], copy_b.at[0])
    cp0b.start()
    cp0f.wait()
    snd_f[0] = loc_f[0].astype(_F16)
    cp0b.wait()
    snd_b[0] = loc_b[0].astype(_F16)

    for t in range(N_DEV - 1):
        b = t % 2
        nxt = (t + 1) % 2
        recv_f_chunk = (my_pos - t - 1) % N_DEV
        recv_b_chunk = (my_pos + t + 1) % N_DEV
        if t >= 2:
            pl.semaphore_wait(rs_credit_f, 1)
        rdma_f = pltpu.make_async_remote_copy(
            src_ref=snd_f.at[b], dst_ref=rcv_f.at[b],
            send_sem=rs_send_f.at[b], recv_sem=rs_recv_f.at[b],
            device_id=right, device_id_type=_LOGICAL)
        rdma_f.start()
        if t >= 2:
            pl.semaphore_wait(rs_credit_b, 1)
        rdma_b = pltpu.make_async_remote_copy(
            src_ref=snd_b.at[b], dst_ref=rcv_b.at[b],
            send_sem=rs_send_b.at[b], recv_sem=rs_recv_b.at[b],
            device_id=left, device_id_type=_LOGICAL)
        rdma_b.start()
        cp_f = pltpu.make_async_copy(
            partial_ref.at[rows(recv_f_chunk), fcols], loc_f.at[nxt],
            copy_f.at[nxt])
        cp_f.start()
        cp_b = pltpu.make_async_copy(
            partial_ref.at[rows(recv_b_chunk), bcols], loc_b.at[nxt],
            copy_b.at[nxt])
        cp_b.start()
        rdma_f.wait()
        if 1 <= t <= N_DEV - 3:
            pl.semaphore_signal(rs_credit_f, inc=1, device_id=left,
                                device_id_type=_LOGICAL)
        rdma_b.wait()
        if 1 <= t <= N_DEV - 3:
            pl.semaphore_signal(rs_credit_b, inc=1, device_id=right,
                                device_id_type=_LOGICAL)
        cp_f.wait()
        cp_b.wait()
        if t < N_DEV - 2:
            snd_f[nxt] = (rcv_f[b].astype(jnp.float32)
                          + loc_f[nxt]).astype(_F16)
            snd_b[nxt] = (rcv_b[b].astype(jnp.float32)
                          + loc_b[nxt]).astype(_F16)

    last = (N_DEV - 2) % 2
    own_f = (my_pos + 1) % N_DEV
    own_b = (my_pos - 1) % N_DEV
    y = (rcv_f[last].astype(jnp.float32) + loc_f[(N_DEV - 1) % 2]) * sc
    g = y * (1.0 / (1.0 + jnp.exp(-y)))
    ag_f[0] = g.astype(_F16)
    stg_f[1] = g
    st_f = pltpu.make_async_copy(
        stg_f.at[1], out_ref.at[rows(own_f), fcols], store_f.at[1])
    st_f.start()
    y = (rcv_b[last].astype(jnp.float32) + loc_b[(N_DEV - 1) % 2]) * sc
    g = y * (1.0 / (1.0 + jnp.exp(-y)))
    ag_b[0] = g.astype(_F16)
    stg_b[1] = g
    st_b = pltpu.make_async_copy(
        stg_b.at[1], out_ref.at[rows(own_b), bcols], store_b.at[1])
    st_b.start()
    prev_st_f = {0: None, 1: st_f}
    prev_st_b = {0: None, 1: st_b}

    for t in range(N_DEV - 1):
        a, b = t % 2, (t + 1) % 2
        recv_f_chunk = (my_pos - t) % N_DEV
        recv_b_chunk = (my_pos + t) % N_DEV
        if t >= 2:
            pl.semaphore_wait(ag_credit_f, 1)
        rdma_f = pltpu.make_async_remote_copy(
            src_ref=ag_f.at[a], dst_ref=ag_f.at[b],
            send_sem=ag_send_f.at[b], recv_sem=ag_recv_f.at[b],
            device_id=right, device_id_type=_LOGICAL)
        rdma_f.start()
        if t >= 2:
            pl.semaphore_wait(ag_credit_b, 1)
        rdma_b = pltpu.make_async_remote_copy(
            src_ref=ag_b.at[a], dst_ref=ag_b.at[b],
            send_sem=ag_send_b.at[b], recv_sem=ag_recv_b.at[b],
            device_id=left, device_id_type=_LOGICAL)
        rdma_b.start()
        rdma_f.wait()
        if 1 <= t <= N_DEV - 3:
            pl.semaphore_signal(ag_credit_f, inc=1, device_id=left,
                                device_id_type=_LOGICAL)
        rdma_b.wait()
        if 1 <= t <= N_DEV - 3:
            pl.semaphore_signal(ag_credit_b, inc=1, device_id=right,
                                device_id_type=_LOGICAL)
        slot = b
        if prev_st_f[slot] is not None:
            prev_st_f[slot].wait()
        stg_f[slot] = ag_f[b].astype(jnp.float32)
        st_f = pltpu.make_async_copy(
            stg_f.at[slot], out_ref.at[rows(recv_f_chunk), fcols],
            store_f.at[slot])
        st_f.start()
        prev_st_f[slot] = st_f
        if prev_st_b[slot] is not None:
            prev_st_b[slot].wait()
        stg_b[slot] = ag_b[b].astype(jnp.float32)
        st_b = pltpu.make_async_copy(
            stg_b.at[slot], out_ref.at[rows(recv_b_chunk), bcols],
            store_b.at[slot])
        st_b.start()
        prev_st_b[slot] = st_b

    for slot in (0, 1):
        if prev_st_f[slot] is not None:
            prev_st_f[slot].wait()
        if prev_st_b[slot] is not None:
            prev_st_b[slot].wait()


def kernel(x, w_mat, scale_x, scale_w):
    partial = jnp.dot(x, w_mat, preferred_element_type=jnp.float32)
    sc = (scale_x * scale_w).astype(jnp.float32)
    d = lax.axis_index("i")
    meta = jnp.stack([
        jnp.asarray(_INV, dtype=jnp.int32)[d],
        jnp.asarray(_NEXT, dtype=jnp.int32)[d],
        jnp.asarray(_PREV, dtype=jnp.int32)[d],
    ]).astype(jnp.int32)

    return pl.pallas_call(
        _body,
        out_shape=jax.ShapeDtypeStruct((ROWS, COLS), jnp.float32),
        in_specs=[
            pl.BlockSpec(memory_space=pltpu.SMEM),
            pl.BlockSpec(memory_space=pltpu.SMEM),
            pl.BlockSpec(memory_space=pl.ANY),
        ],
        out_specs=pl.BlockSpec(memory_space=pl.ANY),
        scratch_shapes=[
            pltpu.VMEM((2, CHUNK, HALF), jnp.float32),
            pltpu.VMEM((2, CHUNK, HALF), jnp.float32),
            pltpu.VMEM((2, CHUNK, HALF), _F16),
            pltpu.VMEM((2, CHUNK, HALF), _F16),
            pltpu.VMEM((2, CHUNK, HALF), _F16),
            pltpu.VMEM((2, CHUNK, HALF), _F16),
            pltpu.VMEM((2, CHUNK, HALF), _F16),
            pltpu.VMEM((2, CHUNK, HALF), _F16),
            pltpu.VMEM((2, CHUNK, HALF), jnp.float32),
            pltpu.VMEM((2, CHUNK, HALF), jnp.float32),
            pltpu.SemaphoreType.DMA((2,)),
            pltpu.SemaphoreType.DMA((2,)),
            pltpu.SemaphoreType.DMA((2,)),
            pltpu.SemaphoreType.DMA((2,)),
            pltpu.SemaphoreType.DMA((2,)),
            pltpu.SemaphoreType.DMA((2,)),
            pltpu.SemaphoreType.DMA((2,)),
            pltpu.SemaphoreType.DMA((2,)),
            pltpu.SemaphoreType.DMA((2,)),
            pltpu.SemaphoreType.DMA((2,)),
            pltpu.SemaphoreType.DMA((2,)),
            pltpu.SemaphoreType.DMA((2,)),
            pltpu.SemaphoreType.REGULAR,
            pltpu.SemaphoreType.REGULAR,
            pltpu.SemaphoreType.REGULAR,
            pltpu.SemaphoreType.REGULAR,
        ],
        compiler_params=pltpu.CompilerParams(collective_id=0),
    )(meta, sc, partial)


# device time: 929299 ns/iter; 3.3580x vs baseline; 1.0681x over previous
import jax
import jax.numpy as jnp
from jax import lax
from jax.experimental import pallas as pl
from jax.experimental.pallas import tpu as pltpu

N_DEV = 32
ROWS = 4096
K = 128
COLS = 8192
NSTREAM = 4
Q = COLS // NSTREAM
CHUNK = ROWS // N_DEV

_PLANE = [(0, 0), (1, 0), (1, 1), (0, 1), (0, 2), (1, 2), (1, 3), (0, 3)]
_SNAKE = [(x, y, z) for z in range(4) for (x, y) in _PLANE]
_C16 = [(0, 0), (0, 1), (0, 2), (0, 3), (1, 3), (1, 2), (1, 1), (2, 1),
        (2, 2), (2, 3), (3, 3), (3, 2), (3, 1), (3, 0), (2, 0), (1, 0)]
_CYCLE = [(0, y, z) for (y, z) in _C16] + \
         [(1, y, z) for (y, z) in reversed(_C16)]
assert len(set(_CYCLE)) == N_DEV
for _i in range(N_DEV):
    _a, _b = _CYCLE[_i], _CYCLE[(_i + 1) % N_DEV]
    assert sum(abs(p - q) for p, q in zip(_a, _b)) == 1, (_a, _b)
_PERM = [_SNAKE.index(c) for c in _CYCLE]
_INV = [0] * N_DEV
for _p, _l in enumerate(_PERM):
    _INV[_l] = _p
_NEXT = [_PERM[(_INV[l] + 1) % N_DEV] for l in range(N_DEV)]
_PREV = [_PERM[(_INV[l] - 1) % N_DEV] for l in range(N_DEV)]

_LOGICAL = pl.DeviceIdType.LOGICAL
_BF16 = jnp.bfloat16
_DIRN = (1, 1, -1, -1)


def _body(meta_ref, sc_ref, x_ref, w_ref, out_ref,
          xb, wb, rcv, snd, ag, stg,
          rs_send, rs_recv, ag_send, ag_recv, store,
          rs_cr0, rs_cr1, rs_cr2, rs_cr3,
          ag_cr0, ag_cr1, ag_cr2, ag_cr3):
    my_pos = meta_ref[0]
    right = meta_ref[1]
    left = meta_ref[2]
    sc = sc_ref[0]
    rs_cr = (rs_cr0, rs_cr1, rs_cr2, rs_cr3)
    ag_cr = (ag_cr0, ag_cr1, ag_cr2, ag_cr3)
    tgt = tuple(right if d > 0 else left for d in _DIRN)
    peer = tuple(left if d > 0 else right for d in _DIRN)

    def rows(i):
        return pl.ds(i * CHUNK, CHUNK)

    def qcols(s):
        return pl.ds(s * Q, Q)

    def contrib(s, c):
        return jnp.dot(xb[rows(c), :], wb[:, s * Q:(s + 1) * Q],
                       preferred_element_type=jnp.float32)

    xb[...] = x_ref[...].astype(_BF16)
    wb[...] = w_ref[...].astype(_BF16)

    barrier = pltpu.get_barrier_semaphore()
    pl.semaphore_signal(barrier, inc=1, device_id=left,
                        device_id_type=_LOGICAL)
    pl.semaphore_signal(barrier, inc=1, device_id=right,
                        device_id_type=_LOGICAL)
    pl.semaphore_wait(barrier, 2)

    for s in range(NSTREAM):
        snd[s, 0] = contrib(s, my_pos).astype(_BF16)

    def rs_step(t, slot, nxt, may_wait_cr, may_sig_cr, stage=True):
        rdmas = []
        for s in range(NSTREAM):
            if may_wait_cr is True:
                pl.semaphore_wait(rs_cr[s], 1)
            elif may_wait_cr is not False:
                @pl.when(may_wait_cr)
                def _(s=s):
                    pl.semaphore_wait(rs_cr[s], 1)
            r = pltpu.make_async_remote_copy(
                src_ref=snd.at[s, slot], dst_ref=rcv.at[s, slot],
                send_sem=rs_send.at[s, slot], recv_sem=rs_recv.at[s, slot],
                device_id=tgt[s], device_id_type=_LOGICAL)
            r.start()
            rdmas.append(r)
        cons = [contrib(s, (my_pos - _DIRN[s] * (t + 1)) % N_DEV)
                for s in range(NSTREAM)]
        for s in range(NSTREAM):
            rdmas[s].wait()
            if stage:
                snd[s, nxt] = (rcv[s, slot].astype(jnp.float32)
                               + cons[s]).astype(_BF16)
                if may_sig_cr is True:
                    pl.semaphore_signal(rs_cr[s], inc=1, device_id=peer[s],
                                        device_id_type=_LOGICAL)
                elif may_sig_cr is not False:
                    @pl.when(may_sig_cr)
                    def _(s=s):
                        pl.semaphore_signal(rs_cr[s], inc=1,
                                            device_id=peer[s],
                                            device_id_type=_LOGICAL)
        return cons

    def rs_double(k, _):
        t0 = 2 * k
        rs_step(t0, 0, 1, k >= 1, k <= 14)
        rs_step(t0 + 1, 1, 0, k >= 1, k <= 13)
        return _

    lax.fori_loop(0, (N_DEV - 2) // 2, rs_double, 0)
    last = rs_step(N_DEV - 2, 0, 1, True, False, stage=False)

    for s in range(NSTREAM):
        own = (my_pos + _DIRN[s]) % N_DEV
        y = (rcv[s, 0].astype(jnp.float32) + last[s]) * sc
        g = y * (1.0 / (1.0 + jnp.exp(-y)))
        ag[s, 0] = g.astype(_BF16)
        stg[s, 0] = g
        st = pltpu.make_async_copy(
            stg.at[s, 0], out_ref.at[rows(own), qcols(s)], store.at[s, 0])
        st.start()

    def ag_step(t, a, b, may_wait_cr, may_sig_cr, may_wait_st):
        rdmas = []
        for s in range(NSTREAM):
            if may_wait_cr is True:
                pl.semaphore_wait(ag_cr[s], 1)
            elif may_wait_cr is not False:
                @pl.when(may_wait_cr)
                def _(s=s):
                    pl.semaphore_wait(ag_cr[s], 1)
            r = pltpu.make_async_remote_copy(
                src_ref=ag.at[s, a], dst_ref=ag.at[s, b],
                send_sem=ag_send.at[s, b], recv_sem=ag_recv.at[s, b],
                device_id=tgt[s], device_id_type=_LOGICAL)
            r.start()
            rdmas.append(r)
        for s in range(NSTREAM):
            rdmas[s].wait()
            if may_sig_cr is True:
                pl.semaphore_signal(ag_cr[s], inc=1, device_id=peer[s],
                                    device_id_type=_LOGICAL)
            elif may_sig_cr is not False:
                @pl.when(may_sig_cr)
                def _(s=s):
                    pl.semaphore_signal(ag_cr[s], inc=1, device_id=peer[s],
                                        device_id_type=_LOGICAL)
            if may_wait_st is True:
                pltpu.make_async_copy(
                    stg.at[s, b], out_ref.at[rows(0), qcols(s)],
                    store.at[s, b]).wait()
            elif may_wait_st is not False:
                @pl.when(may_wait_st)
                def _(s=s):
                    pltpu.make_async_copy(
                        stg.at[s, b], out_ref.at[rows(0), qcols(s)],
                        store.at[s, b]).wait()
            ch = (my_pos - _DIRN[s] * t) % N_DEV
            stg[s, b] = ag[s, b].astype(jnp.float32)
            pltpu.make_async_copy(
                stg.at[s, b], out_ref.at[rows(ch), qcols(s)],
                store.at[s, b]).start()

    def ag_double(k, _):
        t0 = 2 * k
        ag_step(t0, 0, 1, k >= 1, 1 <= k, k >= 1)
        ag_step(t0 + 1, 1, 0, k >= 1, True, True)
        return _

    lax.fori_loop(0, (N_DEV - 2) // 2, ag_double, 0)
    ag_step(N_DEV - 2, 0, 1, True, False, True)

    for s in range(NSTREAM):
        for sl in (0, 1):
            pltpu.make_async_copy(
                stg.at[s, sl], out_ref.at[rows(0), qcols(s)],
                store.at[s, sl]).wait()


def kernel(x, w_mat, scale_x, scale_w):
    sc = (scale_x * scale_w).astype(jnp.float32)
    d = lax.axis_index("i")
    meta = jnp.stack([
        jnp.asarray(_INV, dtype=jnp.int32)[d],
        jnp.asarray(_NEXT, dtype=jnp.int32)[d],
        jnp.asarray(_PREV, dtype=jnp.int32)[d],
    ]).astype(jnp.int32)

    return pl.pallas_call(
        _body,
        out_shape=jax.ShapeDtypeStruct((ROWS, COLS), jnp.float32),
        in_specs=[
            pl.BlockSpec(memory_space=pltpu.SMEM),
            pl.BlockSpec(memory_space=pltpu.SMEM),
            pl.BlockSpec(memory_space=pltpu.VMEM),
            pl.BlockSpec(memory_space=pltpu.VMEM),
        ],
        out_specs=pl.BlockSpec(memory_space=pl.ANY),
        scratch_shapes=[
            pltpu.VMEM((ROWS, K), _BF16),
            pltpu.VMEM((K, COLS), _BF16),
            pltpu.VMEM((NSTREAM, 2, CHUNK, Q), _BF16),
            pltpu.VMEM((NSTREAM, 2, CHUNK, Q), _BF16),
            pltpu.VMEM((NSTREAM, 2, CHUNK, Q), _BF16),
            pltpu.VMEM((NSTREAM, 2, CHUNK, Q), jnp.float32),
            pltpu.SemaphoreType.DMA((NSTREAM, 2)),
            pltpu.SemaphoreType.DMA((NSTREAM, 2)),
            pltpu.SemaphoreType.DMA((NSTREAM, 2)),
            pltpu.SemaphoreType.DMA((NSTREAM, 2)),
            pltpu.SemaphoreType.DMA((NSTREAM, 2)),
            pltpu.SemaphoreType.REGULAR,
            pltpu.SemaphoreType.REGULAR,
            pltpu.SemaphoreType.REGULAR,
            pltpu.SemaphoreType.REGULAR,
            pltpu.SemaphoreType.REGULAR,
            pltpu.SemaphoreType.REGULAR,
            pltpu.SemaphoreType.REGULAR,
            pltpu.SemaphoreType.REGULAR,
        ],
        compiler_params=pltpu.CompilerParams(collective_id=0),
    )(meta, sc, x, w_mat)


# device time: 808440 ns/iter; 3.8600x vs baseline; 1.1495x over previous
import jax
import jax.numpy as jnp
from jax import lax
from jax.experimental import pallas as pl
from jax.experimental.pallas import tpu as pltpu

N_DEV = 32
ROWS = 4096
K = 128
COLS = 8192
NSTREAM = 4
Q = COLS // NSTREAM
CHUNK = ROWS // N_DEV

_PLANE = [(0, 0), (1, 0), (1, 1), (0, 1), (0, 2), (1, 2), (1, 3), (0, 3)]
_SNAKE = [(x, y, z) for z in range(4) for (x, y) in _PLANE]
_C16 = [(0, 0), (0, 1), (0, 2), (0, 3), (1, 3), (1, 2), (1, 1), (2, 1),
        (2, 2), (2, 3), (3, 3), (3, 2), (3, 1), (3, 0), (2, 0), (1, 0)]
_CYCLE = [(0, y, z) for (y, z) in _C16] + \
         [(1, y, z) for (y, z) in reversed(_C16)]
assert len(set(_CYCLE)) == N_DEV
for _i in range(N_DEV):
    _a, _b = _CYCLE[_i], _CYCLE[(_i + 1) % N_DEV]
    assert sum(abs(p - q) for p, q in zip(_a, _b)) == 1, (_a, _b)
_PERM = [_SNAKE.index(c) for c in _CYCLE]
_INV = [0] * N_DEV
for _p, _l in enumerate(_PERM):
    _INV[_l] = _p
_NEXT = [_PERM[(_INV[l] + 1) % N_DEV] for l in range(N_DEV)]
_PREV = [_PERM[(_INV[l] - 1) % N_DEV] for l in range(N_DEV)]

_LOGICAL = pl.DeviceIdType.LOGICAL
_BF16 = jnp.bfloat16
_DIRN = (1, 1, -1, -1)


def _maybe(cond, fn):
    if cond is True:
        fn()
    elif cond is not False:
        pl.when(cond)(fn)


def _body(meta_ref, sc_ref, x_ref, w_ref, out_ref,
          xb, wb, rcv, snd, ag, stg, con,
          rs_send, rs_recv, ag_send, ag_recv, store,
          rs_cr0, rs_cr1, rs_cr2, rs_cr3,
          ag_cr0, ag_cr1, ag_cr2, ag_cr3):
    my_pos = meta_ref[0]
    right = meta_ref[1]
    left = meta_ref[2]
    sc = sc_ref[0]
    rs_cr = (rs_cr0, rs_cr1, rs_cr2, rs_cr3)
    ag_cr = (ag_cr0, ag_cr1, ag_cr2, ag_cr3)
    tgt = tuple(right if d > 0 else left for d in _DIRN)
    peer = tuple(left if d > 0 else right for d in _DIRN)

    def rows(i):
        return pl.ds(i * CHUNK, CHUNK)

    def qcols(s):
        return pl.ds(s * Q, Q)

    def contrib(s, c):
        return jnp.dot(xb[rows(c), :], wb[:, s * Q:(s + 1) * Q],
                       preferred_element_type=jnp.float32)

    def rs_desc(s, slot):
        return pltpu.make_async_remote_copy(
            src_ref=snd.at[s, slot], dst_ref=rcv.at[s, slot],
            send_sem=rs_send.at[s, slot], recv_sem=rs_recv.at[s, slot],
            device_id=tgt[s], device_id_type=_LOGICAL)

    def ag_desc(s, a, b):
        return pltpu.make_async_remote_copy(
            src_ref=ag.at[s, a], dst_ref=ag.at[s, b],
            send_sem=ag_send.at[s, a], recv_sem=ag_recv.at[s, b],
            device_id=tgt[s], device_id_type=_LOGICAL)

    def st_desc(s, slot, ch):
        return pltpu.make_async_copy(
            stg.at[s, slot], out_ref.at[rows(ch), qcols(s)],
            store.at[s, slot])

    xb[...] = x_ref[...].astype(_BF16)
    wb[...] = w_ref[...].astype(_BF16)

    barrier = pltpu.get_barrier_semaphore()
    pl.semaphore_signal(barrier, inc=1, device_id=left,
                        device_id_type=_LOGICAL)
    pl.semaphore_signal(barrier, inc=1, device_id=right,
                        device_id_type=_LOGICAL)
    pl.semaphore_wait(barrier, 2)

    for s in range(NSTREAM):
        snd[s, 0] = contrib(s, my_pos).astype(_BF16)
        rs_desc(s, 0).start()
        con[s, 0] = contrib(s, (my_pos - _DIRN[s]) % N_DEV)

    def rs_iter(t, slot, nxt, wait_cr, sig_cr, issue=True):
        for s in range(NSTREAM):
            rs_desc(s, slot).wait_recv()
            if issue:
                snd[s, nxt] = (rcv[s, slot].astype(jnp.float32)
                               + con[s, slot]).astype(_BF16)
                _maybe(sig_cr, lambda s=s: pl.semaphore_signal(
                    rs_cr[s], inc=1, device_id=peer[s],
                    device_id_type=_LOGICAL))
            rs_desc(s, slot).wait_send()
            if issue:
                _maybe(wait_cr,
                       lambda s=s: pl.semaphore_wait(rs_cr[s], 1))
                rs_desc(s, nxt).start()
                con[s, nxt] = contrib(
                    s, (my_pos - _DIRN[s] * (t + 2)) % N_DEV)

    def rs_double(k, c):
        t0 = 2 * k
        rs_iter(t0, 0, 1, k >= 1, True)
        rs_iter(t0 + 1, 1, 0, True, k <= 13)
        return c

    lax.fori_loop(0, (N_DEV - 2) // 2, rs_double, 0)
    rs_iter(N_DEV - 2, 0, 1, False, False, issue=False)

    for s in range(NSTREAM):
        own = (my_pos + _DIRN[s]) % N_DEV
        y = (rcv[s, 0].astype(jnp.float32) + con[s, 0]) * sc
        g = y * (1.0 / (1.0 + jnp.exp(-y)))
        ag[s, 0] = g.astype(_BF16)
        ag_desc(s, 0, 1).start()
        stg[s, 0] = g
        st_desc(s, 0, own).start()

    def ag_iter(t, a, b, wait_cr, sig_cr, wait_st, issue=True):
        for s in range(NSTREAM):
            ag_desc(s, a, b).wait_send()
            _maybe(sig_cr, lambda s=s: pl.semaphore_signal(
                ag_cr[s], inc=1, device_id=peer[s],
                device_id_type=_LOGICAL))
            ag_desc(s, a, b).wait_recv()
            if issue:
                _maybe(wait_cr,
                       lambda s=s: pl.semaphore_wait(ag_cr[s], 1))
                ag_desc(s, b, a).start()
            ch = (my_pos - _DIRN[s] * t) % N_DEV
            _maybe(wait_st, lambda s=s, ch=ch: st_desc(s, b, ch).wait())
            stg[s, b] = ag[s, b].astype(jnp.float32)
            st_desc(s, b, ch).start()

    def ag_double(k, c):
        t0 = 2 * k
        ag_iter(t0, 0, 1, k >= 1, k >= 1, k >= 1)
        ag_iter(t0 + 1, 1, 0, True, True, True)
        return c

    lax.fori_loop(0, (N_DEV - 2) // 2, ag_double, 0)
    ag_iter(N_DEV - 2, 0, 1, False, False, True, issue=False)

    for s in range(NSTREAM):
        for sl in (0, 1):
            st_desc(s, sl, 0).wait()


def kernel(x, w_mat, scale_x, scale_w):
    sc = (scale_x * scale_w).astype(jnp.float32)
    d = lax.axis_index("i")
    meta = jnp.stack([
        jnp.asarray(_INV, dtype=jnp.int32)[d],
        jnp.asarray(_NEXT, dtype=jnp.int32)[d],
        jnp.asarray(_PREV, dtype=jnp.int32)[d],
    ]).astype(jnp.int32)

    return pl.pallas_call(
        _body,
        out_shape=jax.ShapeDtypeStruct((ROWS, COLS), jnp.float32),
        in_specs=[
            pl.BlockSpec(memory_space=pltpu.SMEM),
            pl.BlockSpec(memory_space=pltpu.SMEM),
            pl.BlockSpec(memory_space=pltpu.VMEM),
            pl.BlockSpec(memory_space=pltpu.VMEM),
        ],
        out_specs=pl.BlockSpec(memory_space=pl.ANY),
        scratch_shapes=[
            pltpu.VMEM((ROWS, K), _BF16),
            pltpu.VMEM((K, COLS), _BF16),
            pltpu.VMEM((NSTREAM, 2, CHUNK, Q), _BF16),
            pltpu.VMEM((NSTREAM, 2, CHUNK, Q), _BF16),
            pltpu.VMEM((NSTREAM, 2, CHUNK, Q), _BF16),
            pltpu.VMEM((NSTREAM, 2, CHUNK, Q), jnp.float32),
            pltpu.VMEM((NSTREAM, 2, CHUNK, Q), jnp.float32),
            pltpu.SemaphoreType.DMA((NSTREAM, 2)),
            pltpu.SemaphoreType.DMA((NSTREAM, 2)),
            pltpu.SemaphoreType.DMA((NSTREAM, 2)),
            pltpu.SemaphoreType.DMA((NSTREAM, 2)),
            pltpu.SemaphoreType.DMA((NSTREAM, 2)),
            pltpu.SemaphoreType.REGULAR,
            pltpu.SemaphoreType.REGULAR,
            pltpu.SemaphoreType.REGULAR,
            pltpu.SemaphoreType.REGULAR,
            pltpu.SemaphoreType.REGULAR,
            pltpu.SemaphoreType.REGULAR,
            pltpu.SemaphoreType.REGULAR,
            pltpu.SemaphoreType.REGULAR,
        ],
        compiler_params=pltpu.CompilerParams(collective_id=0),
    )(meta, sc, x, w_mat)
